# baseline (speedup 1.0000x reference)
import functools

import jax
import jax.numpy as jnp
from jax import lax
from jax.experimental import pallas as pl
from jax.experimental.pallas import tpu as pltpu
from jax.experimental.pallas import tpu_sc as plsc

B = 4
H = 8
QLEN = 2048
KLEN = 8192
D = 64
QBKT = 16
KBKT = 64

NW = 32
ROWS_PER_W = B * H * KLEN // NW
CHUNK = 256
NCHUNK = ROWS_PER_W // CHUNK


def _sc_gather_body(idx_hbm, kv_hbm, skv_hbm,
                    idx_raw, idx_adj, rows0, rows1, sg0, sg1, sw0, sw1):
    cid = lax.axis_index("c")
    sid = lax.axis_index("s")
    wid = sid * 2 + cid
    base = wid * ROWS_PER_W
    boff = (wid // H) * KLEN

    pltpu.sync_copy(idx_hbm.at[pl.ds(base, ROWS_PER_W)], idx_raw)

    def lane_body(i, c):
        v = idx_raw[pl.ds(i * 16, 16)]
        idx_adj[pl.ds(i * 16, 16)] = (v & (KLEN - 1)) + boff
        return c

    lax.fori_loop(0, ROWS_PER_W // 16, lane_body, 0)

    rows = (rows0, rows1)
    sg = (sg0, sg1)
    sw = (sw0, sw1)

    def start_gather(c):
        return pltpu.async_copy(
            kv_hbm.at[idx_adj.at[pl.ds(c * CHUNK, CHUNK)]], rows[c % 2], sg[c % 2])

    def start_write(c):
        return pltpu.async_copy(
            rows[c % 2], skv_hbm.at[pl.ds(base + c * CHUNK, CHUNK)], sw[c % 2])

    g = [None] * NCHUNK
    w = [None] * NCHUNK
    g[0] = start_gather(0)
    for c in range(NCHUNK):
        if c + 1 < NCHUNK:
            if c - 1 >= 0:
                w[c - 1].wait()
            g[c + 1] = start_gather(c + 1)
        g[c].wait()
        w[c] = start_write(c)
    w[NCHUNK - 2].wait()
    w[NCHUNK - 1].wait()


@functools.cache
def _sc_gather():
    return pl.kernel(
        _sc_gather_body,
        out_type=jax.ShapeDtypeStruct((B * H * KLEN, 2 * D), jnp.float32),
        mesh=plsc.VectorSubcoreMesh(core_axis_name="c", subcore_axis_name="s"),
        scratch_types=[
            pltpu.VMEM((ROWS_PER_W,), jnp.int32),
            pltpu.VMEM((ROWS_PER_W,), jnp.int32),
            pltpu.VMEM((CHUNK, 2 * D), jnp.float32),
            pltpu.VMEM((CHUNK, 2 * D), jnp.float32),
            pltpu.SemaphoreType.DMA,
            pltpu.SemaphoreType.DMA,
            pltpu.SemaphoreType.DMA,
            pltpu.SemaphoreType.DMA,
        ],
    )


QTILE = 128
KTILE = 512
NTILE = QLEN // QTILE
NEG_INF = float("-inf")


def _tc_attn_body(q_ref, kv_ref, o_ref, acc, mref, lref):
    h = pl.program_id(0)

    @pl.when(h == 0)
    def _init():
        mref[...] = jnp.full_like(mref, NEG_INF)
        lref[...] = jnp.zeros_like(lref)
        acc[...] = jnp.zeros_like(acc)

    rows = lax.broadcasted_iota(jnp.int32, (KTILE, QTILE), 0) // KBKT
    cols = lax.broadcasted_iota(jnp.int32, (KTILE, QTILE), 1) // QBKT
    diag = rows == cols

    for t in range(NTILE):
        q = q_ref[0, pl.ds(t * QTILE, QTILE), :]
        k = kv_ref[pl.ds(t * KTILE, KTILE), :D]
        v = kv_ref[pl.ds(t * KTILE, KTILE), D:]
        s = lax.dot_general(k, q, (((1,), (1,)), ((), ())),
                            preferred_element_type=jnp.float32)
        s = jnp.where(diag, s, NEG_INF)
        m = jnp.max(s, axis=0, keepdims=True)
        p = jnp.exp(s - m)
        l = jnp.sum(p, axis=0, keepdims=True)
        o = lax.dot_general(v, p, (((0,), (0,)), ((), ())),
                            preferred_element_type=jnp.float32)
        sl = pl.ds(t * QTILE, QTILE)
        m_old = mref[:, sl]
        m_new = jnp.maximum(m_old, m)
        alpha = jnp.exp(m_old - m_new)
        beta = jnp.exp(m - m_new)
        acc[:, sl] = acc[:, sl] * alpha + o * beta
        lref[:, sl] = lref[:, sl] * alpha + l * beta
        mref[:, sl] = m_new

    @pl.when(h == H - 1)
    def _fin():
        o_ref[...] = jnp.transpose(acc[...] / lref[...], (1, 0))


@functools.cache
def _tc_attn(b):
    return pl.pallas_call(
        _tc_attn_body,
        grid=(H,),
        in_specs=[
            pl.BlockSpec((1, QLEN, D), lambda h: (lax.rem(h, B), 0, 0)),
            pl.BlockSpec((KLEN, 2 * D), lambda h: (b * H + h, 0)),
        ],
        out_specs=pl.BlockSpec((QLEN, D), lambda h: (0, 0)),
        out_shape=jax.ShapeDtypeStruct((QLEN, D), jnp.float32),
        scratch_shapes=[
            pltpu.VMEM((D, QLEN), jnp.float32),
            pltpu.VMEM((1, QLEN), jnp.float32),
            pltpu.VMEM((1, QLEN), jnp.float32),
        ],
    )


def kernel(query, key, value, s_k_ticker):
    kv = jnp.concatenate([key, value], axis=-1).reshape(B * KLEN, 2 * D)
    idx_flat = s_k_ticker[:B].reshape(-1)
    skv = _sc_gather()(idx_flat, kv)
    return jnp.stack([_tc_attn(b)(query, skv) for b in range(B)])


# speedup vs baseline: 1.1456x; 1.1456x over previous
import functools

import jax
import jax.numpy as jnp
from jax import lax
from jax.experimental import pallas as pl
from jax.experimental.pallas import tpu as pltpu
from jax.experimental.pallas import tpu_sc as plsc

B = 4
H = 8
QLEN = 2048
KLEN = 8192
D = 64
QBKT = 16
KBKT = 64

NW = 32
ROWS_PER_W = H * KLEN // NW
CHUNK = 256
NCHUNK = ROWS_PER_W // CHUNK


def _sc_gather_body(b, idx_hbm, kv_hbm, skv_hbm, idx_raw, idx_adj,
                    rows0, rows1, rows2, sg0, sg1, sg2, sw0, sw1, sw2):
    cid = lax.axis_index("c")
    sid = lax.axis_index("s")
    wid = sid * 2 + cid
    base = wid * ROWS_PER_W
    boff = b * KLEN

    pltpu.sync_copy(idx_hbm.at[pl.ds(base, ROWS_PER_W)], idx_raw)

    def lane_body(i, c):
        v = idx_raw[pl.ds(i * 16, 16)]
        idx_adj[pl.ds(i * 16, 16)] = (v & (KLEN - 1)) + boff
        return c

    lax.fori_loop(0, ROWS_PER_W // 16, lane_body, 0)

    rows = (rows0, rows1, rows2)
    sg = (sg0, sg1, sg2)
    sw = (sw0, sw1, sw2)

    def start_gather(c):
        return pltpu.async_copy(
            kv_hbm.at[idx_adj.at[pl.ds(c * CHUNK, CHUNK)]], rows[c % 3], sg[c % 3])

    def start_write(c):
        return pltpu.async_copy(
            rows[c % 3], skv_hbm.at[pl.ds(base + c * CHUNK, CHUNK)], sw[c % 3])

    g = [None] * NCHUNK
    w = [None] * NCHUNK
    g[0] = start_gather(0)
    g[1] = start_gather(1)
    for c in range(NCHUNK):
        if c + 2 < NCHUNK:
            if c - 1 >= 0:
                w[c - 1].wait()
            g[c + 2] = start_gather(c + 2)
        g[c].wait()
        w[c] = start_write(c)
    for c in range(max(0, NCHUNK - 3), NCHUNK):
        if w[c] is not None:
            w[c].wait()


@functools.cache
def _sc_gather(b):
    return pl.kernel(
        functools.partial(_sc_gather_body, b),
        out_type=jax.ShapeDtypeStruct((H * KLEN, 2 * D), jnp.float32),
        mesh=plsc.VectorSubcoreMesh(core_axis_name="c", subcore_axis_name="s"),
        scratch_types=[
            pltpu.VMEM((ROWS_PER_W,), jnp.int32),
            pltpu.VMEM((ROWS_PER_W,), jnp.int32),
            pltpu.VMEM((CHUNK, 2 * D), jnp.float32),
            pltpu.VMEM((CHUNK, 2 * D), jnp.float32),
            pltpu.VMEM((CHUNK, 2 * D), jnp.float32),
            pltpu.SemaphoreType.DMA,
            pltpu.SemaphoreType.DMA,
            pltpu.SemaphoreType.DMA,
            pltpu.SemaphoreType.DMA,
            pltpu.SemaphoreType.DMA,
            pltpu.SemaphoreType.DMA,
        ],
    )


QTILE = 128
KTILE = 512
NTILE = QLEN // QTILE
NEG_INF = float("-inf")


def _tc_attn_body(q_ref, kv_ref, o_ref, acc, mref, lref):
    h = pl.program_id(0)

    @pl.when(h == 0)
    def _init():
        mref[...] = jnp.full_like(mref, NEG_INF)
        lref[...] = jnp.zeros_like(lref)
        acc[...] = jnp.zeros_like(acc)

    rows = lax.broadcasted_iota(jnp.int32, (KTILE, QTILE), 0) // KBKT
    cols = lax.broadcasted_iota(jnp.int32, (KTILE, QTILE), 1) // QBKT
    diag = rows == cols

    qsel = lax.rem(h, B)
    for t in range(NTILE):
        q = q_ref[qsel, pl.ds(t * QTILE, QTILE), :]
        k = kv_ref[pl.ds(t * KTILE, KTILE), :D]
        v = kv_ref[pl.ds(t * KTILE, KTILE), D:]
        s = lax.dot_general(k, q, (((1,), (1,)), ((), ())),
                            preferred_element_type=jnp.float32)
        s = jnp.where(diag, s, NEG_INF)
        m = jnp.max(s, axis=0, keepdims=True)
        p = jnp.exp(s - m)
        l = jnp.sum(p, axis=0, keepdims=True)
        o = lax.dot_general(v, p, (((0,), (0,)), ((), ())),
                            preferred_element_type=jnp.float32)
        sl = pl.ds(t * QTILE, QTILE)
        m_old = mref[:, sl]
        m_new = jnp.maximum(m_old, m)
        alpha = jnp.exp(m_old - m_new)
        beta = jnp.exp(m - m_new)
        acc[:, sl] = acc[:, sl] * alpha + o * beta
        lref[:, sl] = lref[:, sl] * alpha + l * beta
        mref[:, sl] = m_new

    @pl.when(h == H - 1)
    def _fin():
        o_ref[...] = jnp.transpose(acc[...] / lref[...], (1, 0))


_tc_attn = pl.pallas_call(
    _tc_attn_body,
    grid=(H,),
    in_specs=[
        pl.BlockSpec((B, QLEN, D), lambda h: (0, 0, 0)),
        pl.BlockSpec((KLEN, 2 * D), lambda h: (h, 0)),
    ],
    out_specs=pl.BlockSpec((QLEN, D), lambda h: (0, 0)),
    out_shape=jax.ShapeDtypeStruct((QLEN, D), jnp.float32),
    scratch_shapes=[
        pltpu.VMEM((D, QLEN), jnp.float32),
        pltpu.VMEM((1, QLEN), jnp.float32),
        pltpu.VMEM((1, QLEN), jnp.float32),
    ],
)


def kernel(query, key, value, s_k_ticker):
    kv = jnp.concatenate([key, value], axis=-1).reshape(B * KLEN, 2 * D)
    idx = s_k_ticker[:B]
    skvs = [_sc_gather(b)(idx[b], kv) for b in range(B)]
    return jnp.stack([_tc_attn(query, skv_b) for skv_b in skvs])
